# baseline (device time: 18363 ns/iter reference)
import jax
import jax.numpy as jnp
from jax import lax
from jax.experimental import pallas as pl
from jax.experimental.pallas import tpu as pltpu

T = 512
TG = T // 4
C = TG // 2
V_SHARD = 4096
D = 512


def kernel(ids, E):
    def body(ids_sref, e_ref, out_ref, gbuf, zbuf, sbuf,
             gsem, osem, zsend, zrecv, bsend, brecv):
        my_x = lax.axis_index("x")
        my_y = lax.axis_index("y")
        my_z = lax.axis_index("z")
        my_g = my_x * 2 + my_y
        z_peer = (my_x, my_y, 1 - my_z)
        xy_peer = {
            1: (my_x, 1 - my_y, my_z),
            2: (1 - my_x, my_y, my_z),
            3: (1 - my_x, 1 - my_y, my_z),
        }

        def issue(i, _):
            raw = ids_sref[my_g * TG + i] - my_z * V_SHARD
            idx = jnp.clip(raw, 0, V_SHARD - 1)
            pltpu.make_async_copy(
                e_ref.at[pl.ds(idx, 1), :], gbuf.at[pl.ds(i, 1), :], gsem
            ).start()
            return 0

        lax.fori_loop(0, TG, issue, 0)

        barrier = pltpu.get_barrier_semaphore()
        for dev in [z_peer] + list(xy_peer.values()):
            pl.semaphore_signal(
                barrier, inc=1, device_id=dev,
                device_id_type=pl.DeviceIdType.MESH,
            )
        pl.semaphore_wait(barrier, 4)

        def drain(i, _):
            pltpu.make_async_copy(
                e_ref.at[pl.ds(0, 1), :], gbuf.at[pl.ds(i, 1), :], gsem
            ).wait()
            raw = ids_sref[my_g * TG + i] - my_z * V_SHARD
            invalid = jnp.logical_or(raw < 0, raw >= V_SHARD)

            @pl.when(invalid)
            def _():
                gbuf[pl.ds(i, 1), :] = jnp.zeros((1, D), jnp.float32)

            return 0

        lax.fori_loop(0, TG, drain, 0)

        zx = []
        for c in range(2):
            zbuf[0, c, :, :] = gbuf[pl.ds(c * C, C), :].astype(jnp.bfloat16)
            r = pltpu.make_async_remote_copy(
                src_ref=zbuf.at[0, c], dst_ref=zbuf.at[1, c],
                send_sem=zsend.at[c], recv_sem=zrecv.at[c],
                device_id=z_peer, device_id_type=pl.DeviceIdType.MESH,
            )
            r.start()
            zx.append(r)

        bx = []
        for c in range(2):
            zx[c].wait()
            reduced = (
                gbuf[pl.ds(c * C, C), :]
                + zbuf[1, c, :, :].astype(jnp.float32)
            ).astype(jnp.bfloat16)
            sbuf[c, :, :] = reduced
            pltpu.make_async_copy(
                sbuf.at[c], out_ref.at[pl.ds(my_g * TG + c * C, C), :], osem
            ).start()
            for k in (1, 2, 3):
                r = pltpu.make_async_remote_copy(
                    src_ref=sbuf.at[c],
                    dst_ref=out_ref.at[pl.ds(my_g * TG + c * C, C), :],
                    send_sem=bsend.at[c, k], recv_sem=brecv.at[c, k],
                    device_id=xy_peer[k],
                    device_id_type=pl.DeviceIdType.MESH,
                )
                r.start()
                bx.append(r)

        for c in range(2):
            pltpu.make_async_copy(
                sbuf.at[c], out_ref.at[pl.ds(my_g * TG + c * C, C), :], osem
            ).wait()
        for r in bx:
            r.wait()

    return pl.pallas_call(
        body,
        out_shape=jax.ShapeDtypeStruct((T, D), jnp.bfloat16),
        in_specs=[
            pl.BlockSpec(memory_space=pltpu.MemorySpace.SMEM),
            pl.BlockSpec(memory_space=pl.ANY),
        ],
        out_specs=pl.BlockSpec(memory_space=pl.ANY),
        scratch_shapes=[
            pltpu.VMEM((TG, D), jnp.float32),
            pltpu.VMEM((2, 2, C, D), jnp.bfloat16),
            pltpu.VMEM((2, C, D), jnp.bfloat16),
            pltpu.SemaphoreType.DMA,
            pltpu.SemaphoreType.DMA,
            pltpu.SemaphoreType.DMA((2,)),
            pltpu.SemaphoreType.DMA((2,)),
            pltpu.SemaphoreType.DMA((2, 4)),
            pltpu.SemaphoreType.DMA((2, 4)),
        ],
        compiler_params=pltpu.CompilerParams(collective_id=0),
    )(ids, pltpu.with_memory_space_constraint(E, pltpu.MemorySpace.HBM))
